# baseline (speedup 1.0000x reference)
import functools

import jax
import jax.numpy as jnp
from jax import lax
from jax.experimental import pallas as pl
from jax.experimental.pallas import tpu as pltpu
from jax.experimental.pallas import tpu_sc as plsc

LATENT = 16
NEMB = 8192
F32 = jnp.float32


def _conv1_kernel(p, w, b, out):
  acc = jnp.dot(p[0], w[...], preferred_element_type=F32) + b[...]
  out[0] = jnp.maximum(acc, 0.0)


def _conv1(p1, w9, b):
  return pl.pallas_call(
      _conv1_kernel,
      grid=(4,),
      in_specs=[pl.BlockSpec((1, 12544, 9), lambda n: (n, 0, 0)),
                pl.BlockSpec((9, 32), lambda n: (0, 0)),
                pl.BlockSpec((1, 32), lambda n: (0, 0))],
      out_specs=pl.BlockSpec((1, 12544, 32), lambda n: (n, 0, 0)),
      out_shape=jax.ShapeDtypeStruct((4, 12544, 32), F32),
  )(p1, w9, b)


def _conv23_kernel(p00, p01, p10, p11, w2, b2, w3, b3, out):
  planes = ((p00, p01), (p10, p11))

  def row(i, c):
    cols = []
    for ky in range(3):
      for kx in range(3):
        pr = planes[ky % 2][kx % 2]
        cols.append(pr[0, i + ky // 2, kx // 2:kx // 2 + 56, :])
    patch = jnp.concatenate(cols, axis=1)
    h = jnp.dot(patch, w2[...], preferred_element_type=F32) + b2[...]
    h = jnp.maximum(h, 0.0)
    z = jnp.dot(h, w3[...], preferred_element_type=F32) + b3[...]
    out[0, i] = z
    return c

  lax.fori_loop(0, 56, row, 0)


def _conv23(planes, w2, b2, w3, b3):
  spec_p = pl.BlockSpec((1, 57, 57, 32), lambda n: (n, 0, 0, 0))
  return pl.pallas_call(
      _conv23_kernel,
      grid=(4,),
      in_specs=[spec_p, spec_p, spec_p, spec_p,
                pl.BlockSpec((288, 64), lambda n: (0, 0)),
                pl.BlockSpec((1, 64), lambda n: (0, 0)),
                pl.BlockSpec((64, 16), lambda n: (0, 0)),
                pl.BlockSpec((1, 16), lambda n: (0, 0))],
      out_specs=pl.BlockSpec((1, 56, 56, 16), lambda n: (n, 0, 0, 0)),
      out_shape=jax.ShapeDtypeStruct((4, 56, 56, 16), F32),
  )(*planes, w2, b2, w3, b3)


_VQ_ROWS = 128
_VQ_CHUNK = 512


def _vq_kernel(flat, embt, out):
  f = flat[...]
  best_v = jnp.full((_VQ_ROWS, 1), jnp.inf, F32)
  best_i = jnp.zeros((_VQ_ROWS, 1), jnp.int32)
  for c in range(NEMB // _VQ_CHUNK):
    ec = embt[:, c * _VQ_CHUNK:(c + 1) * _VQ_CHUNK]
    e2 = jnp.sum(ec * ec, axis=0, keepdims=True)
    d = e2 - 2.0 * jnp.dot(f, ec, preferred_element_type=F32)
    m = jnp.min(d, axis=1, keepdims=True)
    iota = lax.broadcasted_iota(jnp.int32, (_VQ_ROWS, _VQ_CHUNK), 1)
    cand = jnp.where(d <= m, iota + c * _VQ_CHUNK, jnp.int32(2**30))
    ci = jnp.min(cand, axis=1, keepdims=True)
    upd = m < best_v
    best_i = jnp.where(upd, ci, best_i)
    best_v = jnp.minimum(best_v, m)
  out[...] = best_i


def _vq_argmin(flat, embt):
  n = flat.shape[0]
  return pl.pallas_call(
      _vq_kernel,
      grid=(n // _VQ_ROWS,),
      in_specs=[pl.BlockSpec((_VQ_ROWS, 16), lambda i: (i, 0)),
                pl.BlockSpec((16, NEMB), lambda i: (0, 0))],
      out_specs=pl.BlockSpec((_VQ_ROWS, 1), lambda i: (i, 0)),
      out_shape=jax.ShapeDtypeStruct((n, 1), jnp.int32),
  )(flat, embt)


def _gather_sc(emb, idx):
  n = idx.shape[0]
  info = plsc.get_sparse_core_info()
  nw = info.num_cores * info.num_subcores
  b_per_w = n // nw
  mesh = plsc.VectorSubcoreMesh(core_axis_name="c", subcore_axis_name="s")

  @functools.partial(
      pl.kernel,
      mesh=mesh,
      out_type=jax.ShapeDtypeStruct((n, LATENT), F32),
      compiler_params=pltpu.CompilerParams(use_tc_tiling_on_sc=False),
      scratch_types=[
          pltpu.VMEM((b_per_w,), jnp.int32),
          pltpu.VMEM((b_per_w, LATENT), F32),
          pltpu.SemaphoreType.DMA,
      ],
  )
  def gather(table_hbm, idx_hbm, out_hbm, idx_v, rows_v, sem):
    wid = lax.axis_index("s") * info.num_cores + lax.axis_index("c")
    base = wid * b_per_w
    pltpu.sync_copy(idx_hbm.at[pl.ds(base, b_per_w)], idx_v)
    pltpu.async_copy(table_hbm.at[idx_v], rows_v, sem).wait()
    pltpu.sync_copy(rows_v, out_hbm.at[pl.ds(base, b_per_w)])

  return gather(emb, idx)


def _loss_kernel(a, b, out):
  d = a[...] - b[...]
  out[0, 0] = jnp.sum(d * d)


def _loss_sum(a, b):
  return pl.pallas_call(
      _loss_kernel,
      in_specs=[pl.BlockSpec(a.shape, lambda: (0, 0)),
                pl.BlockSpec(a.shape, lambda: (0, 0))],
      out_specs=pl.BlockSpec(memory_space=pltpu.SMEM),
      out_shape=jax.ShapeDtypeStruct((1, 1), F32),
  )(a, b)


def _make_convt_kernel(rows, width):

  def kern(xp, w, b, out):

    def row(i, c):
      s0 = xp[0, i, 0:width, :]
      s1 = xp[0, i, 1:width + 1, :]
      s2 = xp[0, i + 1, 0:width, :]
      s3 = xp[0, i + 1, 1:width + 1, :]
      patch = jnp.concatenate([s0, s1, s2, s3], axis=1)
      r = jnp.dot(patch, w[...], preferred_element_type=F32) + b[...]
      out[0, i] = jnp.maximum(r, 0.0)
      return c

    lax.fori_loop(0, rows, row, 0)

  return kern


def _convt(xp, w_all, b_all, rows, width, cout4):
  return pl.pallas_call(
      _make_convt_kernel(rows, width),
      grid=(4,),
      in_specs=[
          pl.BlockSpec((1,) + xp.shape[1:], lambda n: (n, 0, 0, 0)),
          pl.BlockSpec(w_all.shape, lambda n: (0, 0)),
          pl.BlockSpec((1, cout4), lambda n: (0, 0)),
      ],
      out_specs=pl.BlockSpec((1, rows, width, cout4), lambda n: (n, 0, 0, 0)),
      out_shape=jax.ShapeDtypeStruct((4, rows, width, cout4), F32),
  )(xp, w_all, b_all)


def _pack_convt_w(w):
  cin, cout = w.shape[0], w.shape[1]
  m = lambda a, bb: w[:, :, 2 - a, 2 - bb]
  z = jnp.zeros((cin, cout), F32)
  r0 = jnp.concatenate([m(1, 1), m(1, 0), m(0, 1), m(0, 0)], axis=1)
  r1 = jnp.concatenate([z, m(1, 2), z, m(0, 2)], axis=1)
  r2 = jnp.concatenate([z, z, m(2, 1), m(2, 0)], axis=1)
  r3 = jnp.concatenate([z, z, z, m(2, 2)], axis=1)
  return jnp.concatenate([r0, r1, r2, r3], axis=0)


def _interleave(t, rows, width, cout):
  t = t.reshape(4, rows, width, 2, 2, cout)
  t = jnp.transpose(t, (0, 1, 3, 2, 4, 5))
  return t.reshape(4, 2 * rows, 2 * width, cout)


def _conv3f_kernel(xp, w, b, out):

  def row(i, c):
    cols = []
    for ky in range(3):
      for kx in range(3):
        cols.append(xp[0, i + ky, :, kx:kx + 222])
    patch = jnp.concatenate(cols, axis=0)
    v = jnp.dot(w[...], patch, preferred_element_type=F32) + b[...]
    out[0, pl.ds(i, 1), :] = 1.0 / (1.0 + jnp.exp(-v))
    return c

  lax.fori_loop(0, 222, row, 0)


def _conv3f(xp, w, b):
  return pl.pallas_call(
      _conv3f_kernel,
      grid=(4,),
      in_specs=[pl.BlockSpec((1, 224, 32, 224), lambda n: (n, 0, 0, 0)),
                pl.BlockSpec((1, 288), lambda n: (0, 0)),
                pl.BlockSpec((1, 1), lambda n: (0, 0))],
      out_specs=pl.BlockSpec((1, 222, 222), lambda n: (n, 0, 0)),
      out_shape=jax.ShapeDtypeStruct((4, 222, 222), F32),
  )(xp, w, b)


def _parity_planes(x):
  return [x[:, a::2, b::2, :] for a in (0, 1) for b in (0, 1)]


@jax.jit
def kernel(x, enc_w1, enc_b1, enc_w2, enc_b2, enc_w3, enc_b3,
           dec_w1, dec_b1, dec_w2, dec_b2, dec_w3, dec_b3, embeddings):
  xp = jnp.pad(x[:, 0, :, :], ((0, 0), (1, 1), (1, 1)))
  p1 = jnp.stack([xp[:, ky:ky + 223:2, kx:kx + 223:2]
                  for ky in range(3) for kx in range(3)],
                 axis=-1).reshape(4, 12544, 9)
  w1 = jnp.transpose(enc_w1.reshape(32, 9), (1, 0))
  h1 = _conv1(p1, w1, enc_b1.reshape(1, 32)).reshape(4, 112, 112, 32)

  h1p = jnp.pad(h1, ((0, 0), (1, 1), (1, 1), (0, 0)))
  p2 = _parity_planes(h1p)
  w2 = jnp.transpose(enc_w2, (2, 3, 1, 0)).reshape(288, 64)
  w3 = jnp.transpose(enc_w3.reshape(16, 64), (1, 0))
  z = _conv23(p2, w2, enc_b2.reshape(1, 64), w3, enc_b3.reshape(1, 16))

  flat = z.reshape(-1, LATENT)
  idx2 = _vq_argmin(flat, jnp.transpose(embeddings, (1, 0)))
  idx = idx2.reshape(-1)
  q = _gather_sc(embeddings, idx)

  nel = flat.size
  s = _loss_sum(q.reshape(-1, 128), flat.reshape(-1, 128))
  vq_loss = 1.25 * s[0, 0] / nel

  qp = jnp.pad(q.reshape(4, 56, 56, LATENT),
               ((0, 0), (0, 1), (0, 1), (0, 0)))
  t1 = _convt(qp, _pack_convt_w(dec_w1),
              jnp.tile(dec_b1, 4).reshape(1, 256), 56, 56, 256)
  d1 = _interleave(t1, 56, 56, 64)
  d1p = jnp.pad(d1[:, :111, :111, :], ((0, 0), (0, 1), (0, 1), (0, 0)))

  t2 = _convt(d1p, _pack_convt_w(dec_w2),
              jnp.tile(dec_b2, 4).reshape(1, 128), 111, 111, 128)
  d2s = jnp.transpose(t2.reshape(4, 111, 111, 2, 2, 32),
                      (0, 1, 3, 5, 2, 4)).reshape(4, 222, 32, 222)
  d2s = jnp.pad(d2s, ((0, 0), (1, 1), (0, 0), (1, 1)))

  w3f = jnp.transpose(dec_w3[0], (1, 2, 0)).reshape(1, 288)
  out = _conv3f(d2s, w3f, dec_b3.reshape(1, 1))
  x_recon = out[:, None, :, :]
  return (x_recon, vq_loss)


# speedup vs baseline: 1.5226x; 1.5226x over previous
import functools

import jax
import jax.numpy as jnp
from jax import lax
from jax.experimental import pallas as pl
from jax.experimental.pallas import tpu as pltpu
from jax.experimental.pallas import tpu_sc as plsc

LATENT = 16
NEMB = 8192
F32 = jnp.float32


def _encoder_kernel(xq, w1, b1, w2, b2, w3, b3, out, he, ho):

  def conv1_row(r):
    even_cols = []
    odd_cols = []
    for ky in range(3):
      xr = 2 * r + ky
      for kx in range(3):
        even_cols.append(xq[0, kx, pl.ds(xr, 1), 0:56])
      odd_cols.append(xq[0, 2, pl.ds(xr, 1), 0:56])
      odd_cols.append(xq[0, 3, pl.ds(xr, 1), 0:56])
      odd_cols.append(xq[0, 0, pl.ds(xr, 1), 1:57])
    pe = jnp.concatenate(even_cols, axis=0)
    po = jnp.concatenate(odd_cols, axis=0)
    hev = jnp.maximum(jnp.dot(w1[...], pe, preferred_element_type=F32)
                      + b1[...], 0.0)
    hod = jnp.maximum(jnp.dot(w1[...], po, preferred_element_type=F32)
                      + b1[...], 0.0)
    rp = r + 1
    ho[rp, :, 0:56] = hev
    ho[rp, :, 56:57] = jnp.zeros((32, 1), F32)
    he[rp, :, 1:57] = hod
    he[rp, :, 0:1] = jnp.zeros((32, 1), F32)

  def row(i, c):

    @pl.when(i == 0)
    def _():
      he[0] = jnp.zeros((32, 57), F32)
      ho[0] = jnp.zeros((32, 57), F32)

    conv1_row(2 * i)
    conv1_row(2 * i + 1)

    cols = []
    for ky in range(3):
      rp = 2 * i + ky
      cols.append(he[rp, :, 0:56])
      cols.append(ho[rp, :, 0:56])
      cols.append(he[rp, :, 1:57])
    patch = jnp.concatenate(cols, axis=0)
    h = jnp.maximum(jnp.dot(w2[...], patch, preferred_element_type=F32)
                    + b2[...], 0.0)
    z = jnp.dot(w3[...], h, preferred_element_type=F32) + b3[...]
    out[0, i] = z
    return c

  lax.fori_loop(0, 56, row, 0)


def _encoder(xq, w1, b1, w2, b2, w3, b3):
  return pl.pallas_call(
      _encoder_kernel,
      grid=(4,),
      in_specs=[pl.BlockSpec((1, 4, 226, 57), lambda n: (n, 0, 0, 0)),
                pl.BlockSpec((32, 9), lambda n: (0, 0)),
                pl.BlockSpec((32, 1), lambda n: (0, 0)),
                pl.BlockSpec((64, 288), lambda n: (0, 0)),
                pl.BlockSpec((64, 1), lambda n: (0, 0)),
                pl.BlockSpec((16, 64), lambda n: (0, 0)),
                pl.BlockSpec((16, 1), lambda n: (0, 0))],
      out_specs=pl.BlockSpec((1, 56, 16, 56), lambda n: (n, 0, 0, 0)),
      out_shape=jax.ShapeDtypeStruct((4, 56, 16, 56), F32),
      scratch_shapes=[pltpu.VMEM((114, 32, 57), F32),
                      pltpu.VMEM((114, 32, 57), F32)],
  )(xq, w1, b1, w2, b2, w3, b3)


_VQ_ROWS = 128
_VQ_CHUNK = 512


def _vq_kernel(flat, embt, out):
  f = flat[...]
  best_v = jnp.full((_VQ_ROWS, 1), jnp.inf, F32)
  best_i = jnp.zeros((_VQ_ROWS, 1), jnp.int32)
  for c in range(NEMB // _VQ_CHUNK):
    ec = embt[:, c * _VQ_CHUNK:(c + 1) * _VQ_CHUNK]
    e2 = jnp.sum(ec * ec, axis=0, keepdims=True)
    d = e2 - 2.0 * jnp.dot(f, ec, preferred_element_type=F32)
    m = jnp.min(d, axis=1, keepdims=True)
    iota = lax.broadcasted_iota(jnp.int32, (_VQ_ROWS, _VQ_CHUNK), 1)
    cand = jnp.where(d <= m, iota + c * _VQ_CHUNK, jnp.int32(2**30))
    ci = jnp.min(cand, axis=1, keepdims=True)
    upd = m < best_v
    best_i = jnp.where(upd, ci, best_i)
    best_v = jnp.minimum(best_v, m)
  out[...] = best_i


def _vq_argmin(flat, embt):
  n = flat.shape[0]
  return pl.pallas_call(
      _vq_kernel,
      grid=(n // _VQ_ROWS,),
      in_specs=[pl.BlockSpec((_VQ_ROWS, 16), lambda i: (i, 0)),
                pl.BlockSpec((16, NEMB), lambda i: (0, 0))],
      out_specs=pl.BlockSpec((_VQ_ROWS, 1), lambda i: (i, 0)),
      out_shape=jax.ShapeDtypeStruct((n, 1), jnp.int32),
  )(flat, embt)


def _gather_sc(emb, idx):
  n = idx.shape[0]
  info = plsc.get_sparse_core_info()
  nw = info.num_cores * info.num_subcores
  b_per_w = n // nw
  mesh = plsc.VectorSubcoreMesh(core_axis_name="c", subcore_axis_name="s")

  @functools.partial(
      pl.kernel,
      mesh=mesh,
      out_type=jax.ShapeDtypeStruct((n, LATENT), F32),
      compiler_params=pltpu.CompilerParams(use_tc_tiling_on_sc=False),
      scratch_types=[
          pltpu.VMEM((b_per_w,), jnp.int32),
          pltpu.VMEM((b_per_w, LATENT), F32),
          pltpu.SemaphoreType.DMA,
      ],
  )
  def gather(table_hbm, idx_hbm, out_hbm, idx_v, rows_v, sem):
    wid = lax.axis_index("s") * info.num_cores + lax.axis_index("c")
    base = wid * b_per_w
    pltpu.sync_copy(idx_hbm.at[pl.ds(base, b_per_w)], idx_v)
    pltpu.async_copy(table_hbm.at[idx_v], rows_v, sem).wait()
    pltpu.sync_copy(rows_v, out_hbm.at[pl.ds(base, b_per_w)])

  return gather(emb, idx)


def _loss_kernel(a, b, out):
  d = a[...] - b[...]
  out[0, 0] = jnp.sum(d * d)


def _loss_sum(a, b):
  return pl.pallas_call(
      _loss_kernel,
      in_specs=[pl.BlockSpec(a.shape, lambda: (0, 0)),
                pl.BlockSpec(a.shape, lambda: (0, 0))],
      out_specs=pl.BlockSpec(memory_space=pltpu.SMEM),
      out_shape=jax.ShapeDtypeStruct((1, 1), F32),
  )(a, b)


def _make_convt_kernel(rows, width):

  def kern(xp, w, b, out):

    def row(i, c):
      s0 = xp[0, i, 0:width, :]
      s1 = xp[0, i, 1:width + 1, :]
      s2 = xp[0, i + 1, 0:width, :]
      s3 = xp[0, i + 1, 1:width + 1, :]
      patch = jnp.concatenate([s0, s1, s2, s3], axis=1)
      r = jnp.dot(patch, w[...], preferred_element_type=F32) + b[...]
      out[0, i] = jnp.maximum(r, 0.0)
      return c

    lax.fori_loop(0, rows, row, 0)

  return kern


def _convt(xp, w_all, b_all, rows, width, cout4):
  return pl.pallas_call(
      _make_convt_kernel(rows, width),
      grid=(4,),
      in_specs=[
          pl.BlockSpec((1,) + xp.shape[1:], lambda n: (n, 0, 0, 0)),
          pl.BlockSpec(w_all.shape, lambda n: (0, 0)),
          pl.BlockSpec((1, cout4), lambda n: (0, 0)),
      ],
      out_specs=pl.BlockSpec((1, rows, width, cout4), lambda n: (n, 0, 0, 0)),
      out_shape=jax.ShapeDtypeStruct((4, rows, width, cout4), F32),
  )(xp, w_all, b_all)


def _pack_convt_w(w):
  cin, cout = w.shape[0], w.shape[1]
  m = lambda a, bb: w[:, :, 2 - a, 2 - bb]
  z = jnp.zeros((cin, cout), F32)
  r0 = jnp.concatenate([m(1, 1), m(1, 0), m(0, 1), m(0, 0)], axis=1)
  r1 = jnp.concatenate([z, m(1, 2), z, m(0, 2)], axis=1)
  r2 = jnp.concatenate([z, z, m(2, 1), m(2, 0)], axis=1)
  r3 = jnp.concatenate([z, z, z, m(2, 2)], axis=1)
  return jnp.concatenate([r0, r1, r2, r3], axis=0)


def _interleave(t, rows, width, cout):
  t = t.reshape(4, rows, width, 2, 2, cout)
  t = jnp.transpose(t, (0, 1, 3, 2, 4, 5))
  return t.reshape(4, 2 * rows, 2 * width, cout)


def _conv3f_kernel(xp, w, b, out):

  def row(i, c):
    cols = []
    for ky in range(3):
      for kx in range(3):
        cols.append(xp[0, i + ky, :, kx:kx + 222])
    patch = jnp.concatenate(cols, axis=0)
    v = jnp.dot(w[...], patch, preferred_element_type=F32) + b[...]
    out[0, pl.ds(i, 1), :] = 1.0 / (1.0 + jnp.exp(-v))
    return c

  lax.fori_loop(0, 222, row, 0)


def _conv3f(xp, w, b):
  return pl.pallas_call(
      _conv3f_kernel,
      grid=(4,),
      in_specs=[pl.BlockSpec((1, 224, 32, 224), lambda n: (n, 0, 0, 0)),
                pl.BlockSpec((1, 288), lambda n: (0, 0)),
                pl.BlockSpec((1, 1), lambda n: (0, 0))],
      out_specs=pl.BlockSpec((1, 222, 222), lambda n: (n, 0, 0)),
      out_shape=jax.ShapeDtypeStruct((4, 222, 222), F32),
  )(xp, w, b)


@jax.jit
def kernel(x, enc_w1, enc_b1, enc_w2, enc_b2, enc_w3, enc_b3,
           dec_w1, dec_b1, dec_w2, dec_b2, dec_w3, dec_b3, embeddings):
  xp = jnp.pad(x[:, 0, :, :], ((0, 0), (1, 1), (1, 3)))
  xq = jnp.transpose(xp.reshape(4, 226, 57, 4), (0, 3, 1, 2))
  z = _encoder(xq, enc_w1.reshape(32, 9), enc_b1.reshape(32, 1),
               jnp.transpose(enc_w2.reshape(64, 32, 9),
                             (0, 2, 1)).reshape(64, 288),
               enc_b2.reshape(64, 1), enc_w3.reshape(16, 64),
               enc_b3.reshape(16, 1))

  flat = jnp.transpose(z, (0, 1, 3, 2)).reshape(-1, LATENT)
  idx2 = _vq_argmin(flat, jnp.transpose(embeddings, (1, 0)))
  idx = idx2.reshape(-1)
  q = _gather_sc(embeddings, idx)

  nel = flat.size
  s = _loss_sum(q.reshape(-1, 128), flat.reshape(-1, 128))
  vq_loss = 1.25 * s[0, 0] / nel

  qp = jnp.pad(q.reshape(4, 56, 56, LATENT),
               ((0, 0), (0, 1), (0, 1), (0, 0)))
  t1 = _convt(qp, _pack_convt_w(dec_w1),
              jnp.tile(dec_b1, 4).reshape(1, 256), 56, 56, 256)
  d1 = _interleave(t1, 56, 56, 64)
  d1p = jnp.pad(d1[:, :111, :111, :], ((0, 0), (0, 1), (0, 1), (0, 0)))

  t2 = _convt(d1p, _pack_convt_w(dec_w2),
              jnp.tile(dec_b2, 4).reshape(1, 128), 111, 111, 128)
  d2s = jnp.transpose(t2.reshape(4, 111, 111, 2, 2, 32),
                      (0, 1, 3, 5, 2, 4)).reshape(4, 222, 32, 222)
  d2s = jnp.pad(d2s, ((0, 0), (1, 1), (0, 0), (1, 1)))

  w3f = jnp.transpose(dec_w3[0], (1, 2, 0)).reshape(1, 288)
  out = _conv3f(d2s, w3f, dec_b3.reshape(1, 1))
  x_recon = out[:, None, :, :]
  return (x_recon, vq_loss)


# speedup vs baseline: 1.7098x; 1.1230x over previous
import functools

import jax
import jax.numpy as jnp
from jax import lax
from jax.experimental import pallas as pl
from jax.experimental.pallas import tpu as pltpu
from jax.experimental.pallas import tpu_sc as plsc

LATENT = 16
NEMB = 8192
F32 = jnp.float32


def _encoder_kernel(xq, w1, b1, w2, b2, w3, b3, out, he, ho):

  def conv1_row(r):
    even_cols = []
    odd_cols = []
    for ky in range(3):
      xr = 2 * r + ky
      for kx in range(3):
        even_cols.append(xq[0, kx, pl.ds(xr, 1), 0:56])
      odd_cols.append(xq[0, 2, pl.ds(xr, 1), 0:56])
      odd_cols.append(xq[0, 3, pl.ds(xr, 1), 0:56])
      odd_cols.append(xq[0, 0, pl.ds(xr, 1), 1:57])
    pe = jnp.concatenate(even_cols, axis=0)
    po = jnp.concatenate(odd_cols, axis=0)
    hev = jnp.maximum(jnp.dot(w1[...], pe, preferred_element_type=F32)
                      + b1[...], 0.0)
    hod = jnp.maximum(jnp.dot(w1[...], po, preferred_element_type=F32)
                      + b1[...], 0.0)
    rp = r + 1
    ho[rp, :, 0:56] = hev
    ho[rp, :, 56:57] = jnp.zeros((32, 1), F32)
    he[rp, :, 1:57] = hod
    he[rp, :, 0:1] = jnp.zeros((32, 1), F32)

  def row(i, c):

    @pl.when(i == 0)
    def _():
      he[0] = jnp.zeros((32, 57), F32)
      ho[0] = jnp.zeros((32, 57), F32)

    conv1_row(2 * i)
    conv1_row(2 * i + 1)

    cols = []
    for ky in range(3):
      rp = 2 * i + ky
      cols.append(he[rp, :, 0:56])
      cols.append(ho[rp, :, 0:56])
      cols.append(he[rp, :, 1:57])
    patch = jnp.concatenate(cols, axis=0)
    h = jnp.maximum(jnp.dot(w2[...], patch, preferred_element_type=F32)
                    + b2[...], 0.0)
    z = jnp.dot(w3[...], h, preferred_element_type=F32) + b3[...]
    out[0, i] = z
    return c

  lax.fori_loop(0, 56, row, 0)


def _encoder(xq, w1, b1, w2, b2, w3, b3):
  return pl.pallas_call(
      _encoder_kernel,
      grid=(4,),
      in_specs=[pl.BlockSpec((1, 4, 226, 57), lambda n: (n, 0, 0, 0)),
                pl.BlockSpec((32, 9), lambda n: (0, 0)),
                pl.BlockSpec((32, 1), lambda n: (0, 0)),
                pl.BlockSpec((64, 288), lambda n: (0, 0)),
                pl.BlockSpec((64, 1), lambda n: (0, 0)),
                pl.BlockSpec((16, 64), lambda n: (0, 0)),
                pl.BlockSpec((16, 1), lambda n: (0, 0))],
      out_specs=pl.BlockSpec((1, 56, 16, 56), lambda n: (n, 0, 0, 0)),
      out_shape=jax.ShapeDtypeStruct((4, 56, 16, 56), F32),
      scratch_shapes=[pltpu.VMEM((114, 32, 57), F32),
                      pltpu.VMEM((114, 32, 57), F32)],
  )(xq, w1, b1, w2, b2, w3, b3)


_VQ_ROWS = 128
_VQ_CHUNK = 512


def _vq_kernel(faug, embta, oidx, oloss):
  fb = faug[...]
  best_v = jnp.full((_VQ_ROWS, 1), jnp.inf, F32)
  best_i = jnp.zeros((_VQ_ROWS, 1), jnp.int32)
  iota = lax.broadcasted_iota(jnp.int32, (_VQ_ROWS, _VQ_CHUNK), 1)
  for c in range(NEMB // _VQ_CHUNK):
    ec = embta[:, c * _VQ_CHUNK:(c + 1) * _VQ_CHUNK]
    d = jnp.dot(fb, ec, preferred_element_type=F32)
    m = jnp.min(d, axis=1, keepdims=True)
    cand = jnp.where(d <= m, iota, jnp.int32(2**30))
    ci = jnp.min(cand, axis=1, keepdims=True) + c * _VQ_CHUNK
    upd = m < best_v
    best_i = jnp.where(upd, ci, best_i)
    best_v = jnp.minimum(best_v, m)
  oidx[...] = best_i
  partial = jnp.sum(best_v) + jnp.sum(fb * fb) - F32(_VQ_ROWS)

  @pl.when(pl.program_id(0) == 0)
  def _():
    oloss[0, 0] = partial

  @pl.when(pl.program_id(0) != 0)
  def _():
    oloss[0, 0] = oloss[0, 0] + partial


def _vq_argmin(faug, embta):
  n = faug.shape[0]
  return pl.pallas_call(
      _vq_kernel,
      grid=(n // _VQ_ROWS,),
      in_specs=[pl.BlockSpec((_VQ_ROWS, 17), lambda i: (i, 0)),
                pl.BlockSpec((17, NEMB), lambda i: (0, 0))],
      out_specs=[pl.BlockSpec((_VQ_ROWS, 1), lambda i: (i, 0)),
                 pl.BlockSpec(memory_space=pltpu.SMEM)],
      out_shape=[jax.ShapeDtypeStruct((n, 1), jnp.int32),
                 jax.ShapeDtypeStruct((1, 1), F32)],
  )(faug, embta)


def _gather_sc(emb, idx):
  n = idx.shape[0]
  info = plsc.get_sparse_core_info()
  nw = info.num_cores * info.num_subcores
  b_per_w = n // nw
  mesh = plsc.VectorSubcoreMesh(core_axis_name="c", subcore_axis_name="s")

  @functools.partial(
      pl.kernel,
      mesh=mesh,
      out_type=jax.ShapeDtypeStruct((n, LATENT), F32),
      compiler_params=pltpu.CompilerParams(use_tc_tiling_on_sc=False),
      scratch_types=[
          pltpu.VMEM((b_per_w,), jnp.int32),
          pltpu.VMEM((b_per_w, LATENT), F32),
          pltpu.SemaphoreType.DMA,
      ],
  )
  def gather(table_hbm, idx_hbm, out_hbm, idx_v, rows_v, sem):
    wid = lax.axis_index("s") * info.num_cores + lax.axis_index("c")
    base = wid * b_per_w
    pltpu.sync_copy(idx_hbm.at[pl.ds(base, b_per_w)], idx_v)
    pltpu.async_copy(table_hbm.at[idx_v], rows_v, sem).wait()
    pltpu.sync_copy(rows_v, out_hbm.at[pl.ds(base, b_per_w)])

  return gather(emb, idx)


def _make_convt_kernel(rows, width):

  def kern(xp, w, b, out):

    def row(i, c):
      s0 = xp[0, i, 0:width, :]
      s1 = xp[0, i, 1:width + 1, :]
      s2 = xp[0, i + 1, 0:width, :]
      s3 = xp[0, i + 1, 1:width + 1, :]
      patch = jnp.concatenate([s0, s1, s2, s3], axis=1)
      r = jnp.dot(patch, w[...], preferred_element_type=F32) + b[...]
      out[0, i] = jnp.maximum(r, 0.0)
      return c

    lax.fori_loop(0, rows, row, 0)

  return kern


def _convt(xp, w_all, b_all, rows, width, cout4):
  return pl.pallas_call(
      _make_convt_kernel(rows, width),
      grid=(4,),
      in_specs=[
          pl.BlockSpec((1,) + xp.shape[1:], lambda n: (n, 0, 0, 0)),
          pl.BlockSpec(w_all.shape, lambda n: (0, 0)),
          pl.BlockSpec((1, cout4), lambda n: (0, 0)),
      ],
      out_specs=pl.BlockSpec((1, rows, width, cout4), lambda n: (n, 0, 0, 0)),
      out_shape=jax.ShapeDtypeStruct((4, rows, width, cout4), F32),
  )(xp, w_all, b_all)


def _pack_convt_w(w):
  cin, cout = w.shape[0], w.shape[1]
  m = lambda a, bb: w[:, :, 2 - a, 2 - bb]
  z = jnp.zeros((cin, cout), F32)
  r0 = jnp.concatenate([m(1, 1), m(1, 0), m(0, 1), m(0, 0)], axis=1)
  r1 = jnp.concatenate([z, m(1, 2), z, m(0, 2)], axis=1)
  r2 = jnp.concatenate([z, z, m(2, 1), m(2, 0)], axis=1)
  r3 = jnp.concatenate([z, z, z, m(2, 2)], axis=1)
  return jnp.concatenate([r0, r1, r2, r3], axis=0)


def _interleave(t, rows, width, cout):
  t = t.reshape(4, rows, width, 2, 2, cout)
  t = jnp.transpose(t, (0, 1, 3, 2, 4, 5))
  return t.reshape(4, 2 * rows, 2 * width, cout)


def _conv3f_kernel(xp, w, b, out):

  def rowgrp(g, c):
    grp = []
    for p in range(3):
      i = 3 * g + p
      cols = []
      for ky in range(3):
        for kx in range(3):
          cols.append(xp[0, i + ky, :, kx:kx + 222])
      grp.append(jnp.concatenate(cols, axis=0))
    patch = jnp.concatenate(grp, axis=1)
    v = jnp.dot(w[...], patch, preferred_element_type=F32) + b[...]
    out[0, pl.ds(g, 1), :] = 1.0 / (1.0 + jnp.exp(-v))
    return c

  lax.fori_loop(0, 74, rowgrp, 0)


def _conv3f(xp, w, b):
  return pl.pallas_call(
      _conv3f_kernel,
      grid=(4,),
      in_specs=[pl.BlockSpec((1, 224, 32, 224), lambda n: (n, 0, 0, 0)),
                pl.BlockSpec((1, 288), lambda n: (0, 0)),
                pl.BlockSpec((1, 1), lambda n: (0, 0))],
      out_specs=pl.BlockSpec((1, 74, 666), lambda n: (n, 0, 0)),
      out_shape=jax.ShapeDtypeStruct((4, 74, 666), F32),
  )(xp, w, b)


@jax.jit
def kernel(x, enc_w1, enc_b1, enc_w2, enc_b2, enc_w3, enc_b3,
           dec_w1, dec_b1, dec_w2, dec_b2, dec_w3, dec_b3, embeddings):
  xp = jnp.pad(x[:, 0, :, :], ((0, 0), (1, 1), (1, 3)))
  xq = jnp.transpose(xp.reshape(4, 226, 57, 4), (0, 3, 1, 2))
  z = _encoder(xq, enc_w1.reshape(32, 9), enc_b1.reshape(32, 1),
               jnp.transpose(enc_w2.reshape(64, 32, 9),
                             (0, 2, 1)).reshape(64, 288),
               enc_b2.reshape(64, 1), enc_w3.reshape(16, 64),
               enc_b3.reshape(16, 1))

  flat = jnp.transpose(z, (0, 1, 3, 2)).reshape(-1, LATENT)
  faug = jnp.pad(flat, ((0, 0), (0, 1)), constant_values=1.0)
  embta = jnp.concatenate(
      [-2.0 * jnp.transpose(embeddings, (1, 0)),
       jnp.sum(embeddings * embeddings, axis=1)[None, :]], axis=0)
  idx2, s = _vq_argmin(faug, embta)
  idx = idx2.reshape(-1)
  q = _gather_sc(embeddings, idx)
  vq_loss = 1.25 * s[0, 0] / flat.size

  qp = jnp.pad(q.reshape(4, 56, 56, LATENT),
               ((0, 0), (0, 1), (0, 1), (0, 0)))
  t1 = _convt(qp, _pack_convt_w(dec_w1),
              jnp.tile(dec_b1, 4).reshape(1, 256), 56, 56, 256)
  d1 = _interleave(t1, 56, 56, 64)
  d1p = jnp.pad(d1[:, :111, :111, :], ((0, 0), (0, 1), (0, 1), (0, 0)))

  t2 = _convt(d1p, _pack_convt_w(dec_w2),
              jnp.tile(dec_b2, 4).reshape(1, 128), 111, 111, 128)
  d2s = jnp.transpose(t2.reshape(4, 111, 111, 2, 2, 32),
                      (0, 1, 3, 5, 2, 4)).reshape(4, 222, 32, 222)
  d2s = jnp.pad(d2s, ((0, 0), (1, 1), (0, 0), (1, 1)))

  w3f = jnp.transpose(dec_w3[0], (1, 2, 0)).reshape(1, 288)
  out = _conv3f(d2s, w3f, dec_b3.reshape(1, 1))
  x_recon = out.reshape(4, 1, 222, 222)
  return (x_recon, vq_loss)


# speedup vs baseline: 1.8403x; 1.0764x over previous
import functools

import jax
import jax.numpy as jnp
from jax import lax
from jax.experimental import pallas as pl
from jax.experimental.pallas import tpu as pltpu
from jax.experimental.pallas import tpu_sc as plsc

LATENT = 16
NEMB = 8192
F32 = jnp.float32


def _encoder_kernel(xq, w1, b1, w2, b2, w3, b3, out, he, ho):

  def conv1_row(r):
    even_cols = []
    odd_cols = []
    for ky in range(3):
      xr = 2 * r + ky
      for kx in range(3):
        even_cols.append(xq[0, kx, pl.ds(xr, 1), 0:56])
      odd_cols.append(xq[0, 2, pl.ds(xr, 1), 0:56])
      odd_cols.append(xq[0, 3, pl.ds(xr, 1), 0:56])
      odd_cols.append(xq[0, 0, pl.ds(xr, 1), 1:57])
    pe = jnp.concatenate(even_cols, axis=0)
    po = jnp.concatenate(odd_cols, axis=0)
    hev = jnp.maximum(jnp.dot(w1[...], pe, preferred_element_type=F32)
                      + b1[...], 0.0)
    hod = jnp.maximum(jnp.dot(w1[...], po, preferred_element_type=F32)
                      + b1[...], 0.0)
    rp = r + 1
    ho[rp, :, 0:56] = hev
    ho[rp, :, 56:57] = jnp.zeros((32, 1), F32)
    he[rp, :, 1:57] = hod
    he[rp, :, 0:1] = jnp.zeros((32, 1), F32)

  def row(i, c):

    @pl.when(i == 0)
    def _():
      he[0] = jnp.zeros((32, 57), F32)
      ho[0] = jnp.zeros((32, 57), F32)

    conv1_row(2 * i)
    conv1_row(2 * i + 1)

    cols = []
    for ky in range(3):
      rp = 2 * i + ky
      cols.append(he[rp, :, 0:56])
      cols.append(ho[rp, :, 0:56])
      cols.append(he[rp, :, 1:57])
    patch = jnp.concatenate(cols, axis=0)
    h = jnp.maximum(jnp.dot(w2[...], patch, preferred_element_type=F32)
                    + b2[...], 0.0)
    z = jnp.dot(w3[...], h, preferred_element_type=F32) + b3[...]
    out[0, i] = z
    return c

  lax.fori_loop(0, 56, row, 0)


def _encoder(xq, w1, b1, w2, b2, w3, b3):
  return pl.pallas_call(
      _encoder_kernel,
      grid=(4,),
      in_specs=[pl.BlockSpec((1, 4, 226, 57), lambda n: (n, 0, 0, 0)),
                pl.BlockSpec((32, 9), lambda n: (0, 0)),
                pl.BlockSpec((32, 1), lambda n: (0, 0)),
                pl.BlockSpec((64, 288), lambda n: (0, 0)),
                pl.BlockSpec((64, 1), lambda n: (0, 0)),
                pl.BlockSpec((16, 64), lambda n: (0, 0)),
                pl.BlockSpec((16, 1), lambda n: (0, 0))],
      out_specs=pl.BlockSpec((1, 56, 16, 56), lambda n: (n, 0, 0, 0)),
      out_shape=jax.ShapeDtypeStruct((4, 56, 16, 56), F32),
      scratch_shapes=[pltpu.VMEM((114, 32, 57), F32),
                      pltpu.VMEM((114, 32, 57), F32)],
  )(xq, w1, b1, w2, b2, w3, b3)


_VQ_ROWS = 128
_VQ_CHUNK = 512


def _vq_kernel(faug, embta, oidx, oloss):
  fb = faug[...]
  best_v = jnp.full((_VQ_ROWS, 1), jnp.inf, F32)
  best_i = jnp.zeros((_VQ_ROWS, 1), jnp.int32)
  iota = lax.broadcasted_iota(jnp.int32, (_VQ_ROWS, _VQ_CHUNK), 1)
  for c in range(NEMB // _VQ_CHUNK):
    ec = embta[:, c * _VQ_CHUNK:(c + 1) * _VQ_CHUNK]
    d = jnp.dot(fb, ec, preferred_element_type=F32)
    m = jnp.min(d, axis=1, keepdims=True)
    cand = jnp.where(d <= m, iota, jnp.int32(2**30))
    ci = jnp.min(cand, axis=1, keepdims=True) + c * _VQ_CHUNK
    upd = m < best_v
    best_i = jnp.where(upd, ci, best_i)
    best_v = jnp.minimum(best_v, m)
  oidx[...] = best_i
  partial = jnp.sum(best_v) + jnp.sum(fb * fb) - F32(_VQ_ROWS)

  @pl.when(pl.program_id(0) == 0)
  def _():
    oloss[0, 0] = partial

  @pl.when(pl.program_id(0) != 0)
  def _():
    oloss[0, 0] = oloss[0, 0] + partial


def _vq_argmin(faug, embta):
  n = faug.shape[0]
  return pl.pallas_call(
      _vq_kernel,
      grid=(n // _VQ_ROWS,),
      in_specs=[pl.BlockSpec((_VQ_ROWS, 17), lambda i: (i, 0)),
                pl.BlockSpec((17, NEMB), lambda i: (0, 0))],
      out_specs=[pl.BlockSpec((_VQ_ROWS, 1), lambda i: (i, 0)),
                 pl.BlockSpec(memory_space=pltpu.SMEM)],
      out_shape=[jax.ShapeDtypeStruct((n, 1), jnp.int32),
                 jax.ShapeDtypeStruct((1, 1), F32)],
  )(faug, embta)


def _gather_sc(emb, idx):
  n = idx.shape[0]
  info = plsc.get_sparse_core_info()
  nw = info.num_cores * info.num_subcores
  b_per_w = n // nw
  mesh = plsc.VectorSubcoreMesh(core_axis_name="c", subcore_axis_name="s")

  @functools.partial(
      pl.kernel,
      mesh=mesh,
      out_type=jax.ShapeDtypeStruct((n, LATENT), F32),
      compiler_params=pltpu.CompilerParams(use_tc_tiling_on_sc=False),
      scratch_types=[
          pltpu.VMEM((b_per_w,), jnp.int32),
          pltpu.VMEM((b_per_w, LATENT), F32),
          pltpu.SemaphoreType.DMA,
      ],
  )
  def gather(table_hbm, idx_hbm, out_hbm, idx_v, rows_v, sem):
    wid = lax.axis_index("s") * info.num_cores + lax.axis_index("c")
    base = wid * b_per_w
    pltpu.sync_copy(idx_hbm.at[pl.ds(base, b_per_w)], idx_v)
    pltpu.async_copy(table_hbm.at[idx_v], rows_v, sem).wait()
    pltpu.sync_copy(rows_v, out_hbm.at[pl.ds(base, b_per_w)])

  return gather(emb, idx)


def _make_convt_kernel(rows, width):

  def kern(xp, w, b, out):

    def row(i, c):
      s0 = xp[0, i, :, 0:width]
      s1 = xp[0, i, :, 1:width + 1]
      s2 = xp[0, i + 1, :, 0:width]
      s3 = xp[0, i + 1, :, 1:width + 1]
      patch = jnp.concatenate([s0, s1, s2, s3], axis=0)
      r = jnp.dot(w[...], patch, preferred_element_type=F32) + b[...]
      out[0, i] = jnp.maximum(r, 0.0)
      return c

    lax.fori_loop(0, rows, row, 0)

  return kern


def _convt(xp, w_all, b_all, rows, width, cout4):
  return pl.pallas_call(
      _make_convt_kernel(rows, width),
      grid=(4,),
      in_specs=[
          pl.BlockSpec((1,) + xp.shape[1:], lambda n: (n, 0, 0, 0)),
          pl.BlockSpec(w_all.shape, lambda n: (0, 0)),
          pl.BlockSpec((cout4, 1), lambda n: (0, 0)),
      ],
      out_specs=pl.BlockSpec((1, rows, cout4, width), lambda n: (n, 0, 0, 0)),
      out_shape=jax.ShapeDtypeStruct((4, rows, cout4, width), F32),
  )(xp, w_all, b_all)


def _pack_convt_w(w):
  cin, cout = w.shape[0], w.shape[1]
  m = lambda a, bb: w[:, :, 2 - a, 2 - bb]
  z = jnp.zeros((cin, cout), F32)
  r0 = jnp.concatenate([m(1, 1), m(1, 0), m(0, 1), m(0, 0)], axis=1)
  r1 = jnp.concatenate([z, m(1, 2), z, m(0, 2)], axis=1)
  r2 = jnp.concatenate([z, z, m(2, 1), m(2, 0)], axis=1)
  r3 = jnp.concatenate([z, z, z, m(2, 2)], axis=1)
  return jnp.concatenate([r0, r1, r2, r3], axis=0)


def _interleave(t, rows, cout, width):
  t = t.reshape(4, rows, 2, 2, cout, width)
  t = jnp.transpose(t, (0, 1, 2, 4, 5, 3))
  return t.reshape(4, 2 * rows, cout, 2 * width)


def _conv3f_kernel(xp, w, b, out):

  def rowgrp(g, c):
    grp = []
    for p in range(3):
      i = 3 * g + p
      cols = []
      for ky in range(3):
        for kx in range(3):
          cols.append(xp[0, i + ky, :, kx:kx + 222])
      grp.append(jnp.concatenate(cols, axis=0))
    patch = jnp.concatenate(grp, axis=1)
    v = jnp.dot(w[...], patch, preferred_element_type=F32) + b[...]
    out[0, pl.ds(g, 1), :] = 1.0 / (1.0 + jnp.exp(-v))
    return c

  lax.fori_loop(0, 74, rowgrp, 0)


def _conv3f(xp, w, b):
  return pl.pallas_call(
      _conv3f_kernel,
      grid=(4,),
      in_specs=[pl.BlockSpec((1, 224, 32, 224), lambda n: (n, 0, 0, 0)),
                pl.BlockSpec((1, 288), lambda n: (0, 0)),
                pl.BlockSpec((1, 1), lambda n: (0, 0))],
      out_specs=pl.BlockSpec((1, 74, 666), lambda n: (n, 0, 0)),
      out_shape=jax.ShapeDtypeStruct((4, 74, 666), F32),
  )(xp, w, b)


@jax.jit
def kernel(x, enc_w1, enc_b1, enc_w2, enc_b2, enc_w3, enc_b3,
           dec_w1, dec_b1, dec_w2, dec_b2, dec_w3, dec_b3, embeddings):
  xp = jnp.pad(x[:, 0, :, :], ((0, 0), (1, 1), (1, 3)))
  xq = jnp.transpose(xp.reshape(4, 226, 57, 4), (0, 3, 1, 2))
  z = _encoder(xq, enc_w1.reshape(32, 9), enc_b1.reshape(32, 1),
               jnp.transpose(enc_w2.reshape(64, 32, 9),
                             (0, 2, 1)).reshape(64, 288),
               enc_b2.reshape(64, 1), enc_w3.reshape(16, 64),
               enc_b3.reshape(16, 1))

  flat = jnp.transpose(z, (0, 1, 3, 2)).reshape(-1, LATENT)
  faug = jnp.pad(flat, ((0, 0), (0, 1)), constant_values=1.0)
  embta = jnp.concatenate(
      [-2.0 * jnp.transpose(embeddings, (1, 0)),
       jnp.sum(embeddings * embeddings, axis=1)[None, :]], axis=0)
  idx2, s = _vq_argmin(faug, embta)
  idx = idx2.reshape(-1)
  q = _gather_sc(embeddings, idx)
  vq_loss = 1.25 * s[0, 0] / flat.size

  qt = jnp.pad(jnp.transpose(q.reshape(4, 56, 56, LATENT), (0, 1, 3, 2)),
               ((0, 0), (0, 1), (0, 0), (0, 1)))
  t1 = _convt(qt, jnp.transpose(_pack_convt_w(dec_w1), (1, 0)),
              jnp.tile(dec_b1, 4).reshape(256, 1), 56, 56, 256)
  d1 = _interleave(t1, 56, 64, 56)
  d1p = jnp.pad(d1[:, :111, :, :111], ((0, 0), (0, 1), (0, 0), (0, 1)))

  t2 = _convt(d1p, jnp.transpose(_pack_convt_w(dec_w2), (1, 0)),
              jnp.tile(dec_b2, 4).reshape(128, 1), 111, 111, 128)
  d2s = jnp.pad(_interleave(t2, 111, 32, 111),
                ((0, 0), (1, 1), (0, 0), (1, 1)))

  w3f = jnp.transpose(dec_w3[0], (1, 2, 0)).reshape(1, 288)
  out = _conv3f(d2s, w3f, dec_b3.reshape(1, 1))
  x_recon = out.reshape(4, 1, 222, 222)
  return (x_recon, vq_loss)
